# baseline (speedup 1.0000x reference)
import functools

import jax
import jax.numpy as jnp
from jax import lax
from jax.experimental import pallas as pl
from jax.experimental.pallas import tpu as pltpu
from jax.experimental.pallas import tpu_sc as plsc

_B = 16
_P = 4096
_S = 256
_L = 16
_NS = 16
_WPT = _B * _P * _S // 32
_CW = 32768
_NCHUNK = _WPT // _CW
_GROUPS = _CW // (_L * _S)

_sc_mesh = plsc.VectorSubcoreMesh(core_axis_name="c", subcore_axis_name="s")


@functools.partial(
    pl.kernel,
    mesh=_sc_mesh,
    compiler_params=pltpu.CompilerParams(needs_layout_passes=False),
    out_type=jax.ShapeDtypeStruct((_B, _S), jnp.float32),
    scratch_types=[
        pltpu.VMEM((_CW,), jnp.int32),
        pltpu.VMEM((_CW,), jnp.int32),
        pltpu.VMEM((_L * _S,), jnp.int32),
        pltpu.VMEM((_S,), jnp.int32),
        pltpu.VMEM((_S,), jnp.int32),
        pltpu.VMEM((_S,), jnp.float32),
        pltpu.VMEM_SHARED((_NS, _S), jnp.int32),
        pltpu.SemaphoreType.DMA,
        pltpu.SemaphoreType.DMA,
    ],
)
def _probs_sc(x_hbm, out_hbm, buf0, buf1, h2d, hrow, prt, prow, shared, s0, s1):
    c = lax.axis_index("c")
    s = lax.axis_index("s")
    wid = c * _NS + s
    base = wid * _WPT

    iota = lax.iota(jnp.int32, _L)
    lane_off = iota * _S
    ones = jnp.ones((_L,), jnp.int32)
    zeros = jnp.zeros((_L,), jnp.int32)
    perms = [jnp.bitwise_and(iota + j, _L - 1) for j in range(_L)]

    def zbody(j, carry):
        h2d[pl.ds(j * _L, _L)] = zeros
        return carry

    lax.fori_loop(0, (_L * _S) // _L, zbody, 0)

    def start(cidx, buf, sem):
        pltpu.async_copy(x_hbm.at[pl.ds(base + cidx * _CW, _CW)], buf, sem)

    def wait(buf, sem):
        pltpu.make_async_copy(x_hbm.at[pl.ds(0, _CW)], buf, sem).wait()

    def process(buf):
        def gbody(g, carry):
            kvec0 = iota * _S + g * (_L * _S)

            def kbody(k, accs):
                acc, kvec = accs
                for j in range(_L):
                    acc = acc + plsc.load_gather(buf, [kvec + perms[j]])
                return (acc, kvec + _L)

            acc, _ = lax.fori_loop(0, _L, kbody, (zeros, kvec0))
            plsc.addupdate_scatter(h2d, [acc + lane_off], ones, mask=acc < _S)
            return carry

        lax.fori_loop(0, _GROUPS, gbody, 0)

    start(0, buf0, s0)
    start(1, buf1, s1)

    def cbody(cpair, carry):
        cidx = cpair * 2
        wait(buf0, s0)
        process(buf0)

        @pl.when(cidx + 2 < _NCHUNK)
        def _():
            start(cidx + 2, buf0, s0)

        wait(buf1, s1)
        process(buf1)

        @pl.when(cidx + 3 < _NCHUNK)
        def _():
            start(cidx + 3, buf1, s1)

        return carry

    lax.fori_loop(0, _NCHUNK // 2, cbody, 0)

    def rbody(j, carry):
        acc = h2d[pl.ds(j * _L, _L)]
        for l in range(1, _L):
            acc = acc + h2d[pl.ds(l * _S + j * _L, _L)]
        hrow[pl.ds(j * _L, _L)] = acc
        return carry

    lax.fori_loop(0, _S // _L, rbody, 0)

    pltpu.sync_copy(hrow, shared.at[s])
    plsc.subcore_barrier()

    @pl.when(lax.rem(s, 2) == 0)
    def _():
        pltpu.sync_copy(shared.at[s + 1], prt)

        def mbody(j, tot):
            v = hrow[pl.ds(j * _L, _L)] + prt[pl.ds(j * _L, _L)]
            vf = v.astype(jnp.float32)
            prow[pl.ds(j * _L, _L)] = vf
            return tot + vf

        tot_vec = lax.fori_loop(0, _S // _L, mbody, jnp.zeros((_L,), jnp.float32))
        total = lax.broadcast_in_dim(jnp.sum(tot_vec), (_L,), ())

        def nbody(j, carry):
            prow[pl.ds(j * _L, _L)] = prow[pl.ds(j * _L, _L)] / total
            return carry

        lax.fori_loop(0, _S // _L, nbody, 0)
        row = c * (_NS // 2) + lax.div(s, 2)
        pltpu.sync_copy(prow, out_hbm.at[row])


def kernel(inputs):
    x = inputs.reshape(_B * _P * _S)
    probs = _probs_sc(x)
    return ((probs,),)
